# Initial kernel scaffold; baseline (speedup 1.0000x reference)
import functools

import jax
import jax.numpy as jnp
from jax import lax
from jax.experimental import pallas as pl
from jax.experimental.pallas import tpu as pltpu
from jax.experimental.pallas import tpu_sc as plsc

N = 10000
E = 320000
D = 128
NUM_SAMPLES = 5

NC = 2
NS = 16
NW = NC * NS
EPW = E // NW
K = 100
NCHUNK = EPW // K
ROWS_PT = N // NS

_mesh = plsc.VectorSubcoreMesh(core_axis_name="c", subcore_axis_name="s")


@functools.partial(
    pl.kernel,
    out_type=jax.ShapeDtypeStruct((NC, N, D), jnp.float32),
    mesh=_mesh,
    scratch_types=[
        pltpu.VMEM((NCHUNK, K), jnp.int32),
        pltpu.VMEM((NCHUNK, K), jnp.int32),
        pltpu.VMEM((NCHUNK, K), jnp.float32),
        pltpu.VMEM((K, D), jnp.float32),
        pltpu.VMEM((K, D), jnp.float32),
        pltpu.VMEM((128, D), jnp.float32),
        pltpu.VMEM_SHARED((N, D), jnp.float32),
        pltpu.SemaphoreType.DMA,
        pltpu.SemaphoreType.DMA,
    ],
)
def _sc_spmm(x_hbm, row_hbm, col_hbm, val_hbm, out_hbm,
             colv, rowv, valv, bufa, bufb, zbuf, acc, sema, semb):
    cid = lax.axis_index("c")
    sid = lax.axis_index("s")
    wid = sid * NC + cid

    def zero_body(i, _):
        for j in range(D // 16):
            zbuf[i, pl.ds(j * 16, 16)] = jnp.zeros((16,), jnp.float32)
        return _
    lax.fori_loop(0, 128, zero_body, None)
    for q in range(5):
        pltpu.sync_copy(zbuf.at[pl.ds(0, 125)],
                        acc.at[pl.ds(sid * ROWS_PT + 125 * q, 125)])

    pltpu.sync_copy(col_hbm.at[wid], colv)
    pltpu.sync_copy(row_hbm.at[wid], rowv)
    pltpu.sync_copy(val_hbm.at[wid], valv)

    plsc.subcore_barrier()

    def scale_scatter(c, buf):
        def e_body(e, _):
            v = valv[c, e]
            for j in range(D // 16):
                sl = pl.ds(j * 16, 16)
                buf[e, sl] = buf[e, sl] * v
            return _
        lax.fori_loop(0, K, e_body, None)
        pltpu.sync_copy(buf, acc.at[rowv.at[c]], add=True)

    pltpu.async_copy(x_hbm.at[colv.at[0]], bufa, sema)

    def chunk_pair(i, _):
        c0 = 2 * i
        c1 = 2 * i + 1
        pltpu.make_async_copy(x_hbm.at[colv.at[c0]], bufa, sema).wait()
        pltpu.async_copy(x_hbm.at[colv.at[c1]], bufb, semb)
        scale_scatter(c0, bufa)

        @pl.when(c1 + 1 < NCHUNK)
        def _start_next():
            pltpu.async_copy(x_hbm.at[colv.at[c1 + 1]], bufa, sema)

        pltpu.make_async_copy(x_hbm.at[colv.at[c1]], bufb, semb).wait()
        scale_scatter(c1, bufb)
        return _
    lax.fori_loop(0, NCHUNK // 2, chunk_pair, None)

    plsc.subcore_barrier()

    for q in range(5):
        rs = pl.ds(sid * ROWS_PT + 125 * q, 125)
        pltpu.sync_copy(acc.at[rs], zbuf.at[pl.ds(0, 125)])
        pltpu.sync_copy(zbuf.at[pl.ds(0, 125)], out_hbm.at[cid, rs])


BN = 1000


def _tc_post_body(p_ref, w_ref, b_ref, g_ref, bt_ref, o_ref):
    s = p_ref[0] + p_ref[1]
    h = jnp.dot(s, w_ref[...], preferred_element_type=jnp.float32) + b_ref[...]
    h2 = jnp.where(h > 0, h, jnp.expm1(h))
    mu = jnp.mean(h2, axis=-1, keepdims=True)
    var = jnp.mean(jnp.square(h2 - mu), axis=-1, keepdims=True)
    hn = (h2 - mu) * lax.rsqrt(var + 1e-5) * g_ref[...] + bt_ref[...]
    o_ref[...] = jnp.broadcast_to(hn[:, None, :], (BN, NUM_SAMPLES, D))


def kernel(adj_indices, adj_values, x, W, b, ln_gamma, ln_beta):
    row3 = adj_indices[0].reshape(NW, NCHUNK, K)
    col3 = adj_indices[1].reshape(NW, NCHUNK, K)
    val3 = adj_values.reshape(NW, NCHUNK, K)

    partials = _sc_spmm(x, row3, col3, val3)

    out = pl.pallas_call(
        _tc_post_body,
        grid=(N // BN,),
        in_specs=[
            pl.BlockSpec((NC, BN, D), lambda i: (0, i, 0)),
            pl.BlockSpec((D, D), lambda i: (0, 0)),
            pl.BlockSpec((1, D), lambda i: (0, 0)),
            pl.BlockSpec((1, D), lambda i: (0, 0)),
            pl.BlockSpec((1, D), lambda i: (0, 0)),
        ],
        out_specs=pl.BlockSpec((BN, NUM_SAMPLES, D), lambda i: (i, 0, 0)),
        out_shape=jax.ShapeDtypeStruct((N, NUM_SAMPLES, D), jnp.float32),
    )(partials, W, b, ln_gamma.reshape(1, D), ln_beta.reshape(1, D))
    return out


# speedup vs baseline: 9.3847x; 9.3847x over previous
import functools

import jax
import jax.numpy as jnp
from jax import lax
from jax.experimental import pallas as pl
from jax.experimental.pallas import tpu as pltpu
from jax.experimental.pallas import tpu_sc as plsc

N = 10000
E = 320000
D = 128
NUM_SAMPLES = 5

NC = 2
NS = 16
NW = NC * NS
EPW = E // NW
K = 80
NCHUNK = EPW // K
ROWS_PT = 624
CP_CHUNKS = tuple((i * 80, 80) for i in range(7)) + ((560, 64),)
TAIL_BASE = NS * ROWS_PT

_mesh = plsc.VectorSubcoreMesh(core_axis_name="c", subcore_axis_name="s")


@functools.partial(
    pl.kernel,
    out_type=jax.ShapeDtypeStruct((NC, N, D), jnp.float32),
    mesh=_mesh,
    scratch_types=[
        [pltpu.VMEM((2, K), jnp.int32) for _ in range(3)],
        [pltpu.VMEM((1, K), jnp.float32) for _ in range(3)],
        [pltpu.VMEM((1, K), jnp.int32) for _ in range(3)],
        [pltpu.VMEM((K, D), jnp.float32) for _ in range(3)],
        pltpu.VMEM_SHARED((N, D), jnp.float32),
        [pltpu.SemaphoreType.DMA for _ in range(3)],
        [pltpu.SemaphoreType.DMA for _ in range(3)],
        [pltpu.SemaphoreType.DMA for _ in range(3)],
    ],
)
def _sc_spmm(x_hbm, edges_hbm, vals_hbm, out_hbm,
             eset, vset, sidx, gbuf, acc, sg, ss, se):
    cid = lax.axis_index("c")
    sid = lax.axis_index("s")
    wid = sid * NC + cid

    def zero_body(i, _):
        for j in range(D // 16):
            gbuf[0][i, pl.ds(j * 16, 16)] = jnp.zeros((16,), jnp.float32)
        return _
    lax.fori_loop(0, K, zero_body, None)
    for off, n in CP_CHUNKS:
        pltpu.sync_copy(gbuf[0].at[pl.ds(0, n)],
                        acc.at[pl.ds(sid * ROWS_PT + off, n)])

    @pl.when(sid == 0)
    def _zero_tail():
        pltpu.sync_copy(gbuf[0].at[pl.ds(0, 16)], acc.at[pl.ds(TAIL_BASE, 16)])

    plsc.subcore_barrier()

    def snap_sidx(j):
        for g in range(K // 16):
            sidx[j][0, pl.ds(g * 16, 16)] = eset[j][0, pl.ds(g * 16, 16)]

    def scale(c, buf, vbuf):
        def g_body(g, _):
            vv = vbuf[0, pl.ds(g * 16, 16)]
            for l in range(16):
                v = vv[l]
                i = g * 16 + l
                for j in range(D // 16):
                    sl = pl.ds(j * 16, 16)
                    buf[i, sl] = buf[i, sl] * v
            return _
        lax.fori_loop(0, K // 16, g_body, None)

    def start_eload(c, j):
        pltpu.async_copy(edges_hbm.at[wid, c], eset[j], se[j])
        pltpu.async_copy(vals_hbm.at[wid, c], vset[j], se[j])

    def wait_eload(c, j):
        pltpu.make_async_copy(edges_hbm.at[wid, c], eset[j], se[j]).wait()
        pltpu.make_async_copy(vals_hbm.at[wid, c], vset[j], se[j]).wait()

    def start_gather(j):
        pltpu.async_copy(x_hbm.at[eset[j].at[1]], gbuf[j], sg[j])

    def wait_gather(j):
        pltpu.make_async_copy(x_hbm.at[eset[j].at[1]], gbuf[j], sg[j]).wait()

    def start_scatter(j):
        pltpu.async_copy(gbuf[j], acc.at[sidx[j].at[0]], ss[j], add=True)

    def wait_scatter(j):
        pltpu.make_async_copy(gbuf[j], acc.at[sidx[j].at[0]], ss[j]).wait()

    pltpu.sync_copy(edges_hbm.at[wid, 0], eset[0])
    pltpu.sync_copy(vals_hbm.at[wid, 0], vset[0])
    start_eload(1, 1)
    start_eload(2, 2)
    start_gather(0)

    def peel(c, j, jn):
        wait_eload(c + 1, jn)
        start_gather(jn)
        wait_gather(j)
        scale(c, gbuf[j], vset[j])
        snap_sidx(j)
        start_scatter(j)
        start_eload(c + 3, j)

    peel(0, 0, 1)
    peel(1, 1, 2)

    def triple(k, _):
        base = 3 * k + 2
        for jj in range(3):
            c = base + jj
            j = (2 + jj) % 3
            jn = (j + 1) % 3
            wait_scatter(jn)

            @pl.when(c + 1 < NCHUNK)
            def _next_gather():
                wait_eload(c + 1, jn)
                start_gather(jn)

            wait_gather(j)
            scale(c, gbuf[j], vset[j])
            snap_sidx(j)
            start_scatter(j)

            @pl.when(c + 3 < NCHUNK)
            def _next_eload():
                start_eload(c + 3, j)
        return _
    lax.fori_loop(0, (NCHUNK - 2) // 3, triple, None)

    wait_scatter(0)
    wait_scatter(1)

    plsc.subcore_barrier()

    for off, n in CP_CHUNKS:
        rs = pl.ds(sid * ROWS_PT + off, n)
        pltpu.sync_copy(acc.at[rs], gbuf[0].at[pl.ds(0, n)])
        pltpu.sync_copy(gbuf[0].at[pl.ds(0, n)], out_hbm.at[cid, rs])

    @pl.when(sid == 0)
    def _copy_tail():
        rs = pl.ds(TAIL_BASE, 16)
        pltpu.sync_copy(acc.at[rs], gbuf[0].at[pl.ds(0, 16)])
        pltpu.sync_copy(gbuf[0].at[pl.ds(0, 16)], out_hbm.at[cid, rs])


BN = 1000


def _tc_post_body(p_ref, w_ref, b_ref, g_ref, bt_ref, o_ref):
    s = p_ref[0] + p_ref[1]
    h = jnp.dot(s, w_ref[...], preferred_element_type=jnp.float32) + b_ref[...]
    h2 = jnp.where(h > 0, h, jnp.exp(h) - 1.0)
    mu = jnp.mean(h2, axis=-1, keepdims=True)
    var = jnp.mean(jnp.square(h2 - mu), axis=-1, keepdims=True)
    hn = (h2 - mu) * lax.rsqrt(var + 1e-5) * g_ref[...] + bt_ref[...]
    o_ref[...] = jnp.broadcast_to(hn[:, None, :], (BN, NUM_SAMPLES, D))


def kernel(adj_indices, adj_values, x, W, b, ln_gamma, ln_beta):
    row4 = adj_indices[0].reshape(NW, NCHUNK, 1, K)
    col4 = adj_indices[1].reshape(NW, NCHUNK, 1, K)
    edges = jnp.concatenate([row4, col4], axis=2)
    vals = adj_values.reshape(NW, NCHUNK, 1, K)

    partials = _sc_spmm(x, edges, vals)

    out = pl.pallas_call(
        _tc_post_body,
        grid=(N // BN,),
        in_specs=[
            pl.BlockSpec((NC, BN, D), lambda i: (0, i, 0)),
            pl.BlockSpec((D, D), lambda i: (0, 0)),
            pl.BlockSpec((1, D), lambda i: (0, 0)),
            pl.BlockSpec((1, D), lambda i: (0, 0)),
            pl.BlockSpec((1, D), lambda i: (0, 0)),
        ],
        out_specs=pl.BlockSpec((BN, NUM_SAMPLES, D), lambda i: (i, 0, 0)),
        out_shape=jax.ShapeDtypeStruct((N, NUM_SAMPLES, D), jnp.float32),
    )(partials, W, b, ln_gamma.reshape(1, D), ln_beta.reshape(1, D))
    return out
